# baseline (speedup 1.0000x reference)
import functools

import numpy as np
import jax
import jax.numpy as jnp
from jax import lax
from jax.experimental import pallas as pl
from jax.experimental.pallas import tpu as pltpu
from jax.experimental.pallas import tpu_sc as plsc

H = 224
HH = H // 2
OW = 4 * H
B = 4
C = 192
NPLANES = B * C
OUT_PLANE = HH * OW
NWORKERS = 32
PER_WORKER = NPLANES // NWORKERS
CROWS = 8
NCHUNK = HH // CROWS
CHUNK = CROWS * OW
ROWVREG = OW // 32
IDXWORDS = OUT_PLANE // 2
NCORNER = 4 * HH
CTABWORDS = NCORNER // 2
CORNWORDS = (NCORNER - 1) * 17 + 16


def _pack_pairs(r, c):
    rr = r.astype(np.uint32).reshape(-1, 2, 16)
    cc = c.astype(np.uint32).reshape(-1, 2, 16)
    packed = rr[:, 0] | (cc[:, 0] << 8) | (rr[:, 1] << 16) | (cc[:, 1] << 24)
    return packed.reshape(-1).view(np.int32)


def _build_idx_table() -> np.ndarray:
    lmid = (H - 1) // 2
    r = np.arange(HH)[:, None]
    j = np.arange(OW)[None, :]
    i = (lmid - r) + 0 * j
    el = 2 * r + 1
    hi = i + el
    b1 = 3 * i + el
    b2 = 3 * i + 2 * el
    b3 = 7 * i + 3 * el
    b4 = 7 * i + 4 * el
    body = 4 * H - 4
    k = 5 * i + 2 * el + hi
    in_a = (j < b1) | (j >= b4)
    jm = j - body * (j >= b4)
    in_cr = ~in_a & (j < b2)
    in_b = ~in_a & ~in_cr & (j < b3)
    src_r = np.where(in_a, i,
            np.where(in_cr, j - (2 * i + el),
            np.where(in_b, hi, body - j)))
    src_c = np.where(in_a, np.clip(jm, i, hi),
            np.where(in_cr, hi,
            np.where(in_b, np.clip(k - j, i, hi), i)))
    rep_q0 = in_a & (jm < i)
    rep_q1 = in_a & (jm > hi)
    rep_q2 = in_b & (k - j > hi)
    rep_q3 = in_b & (k - j < i)
    isrep = rep_q0 | rep_q1 | rep_q2 | rep_q3
    q = rep_q1 * 1 + rep_q2 * 2 + rep_q3 * 3
    lane = np.broadcast_to(np.arange(OW)[None, :] % 16, src_r.shape)
    r_field = np.where(isrep, H + q, src_r)
    c_field = np.where(isrep, lane, src_c)
    main = _pack_pairs(r_field.reshape(-1), c_field.reshape(-1))
    rr = np.arange(HH)
    ii = lmid - rr
    hh = HH + rr
    cr = np.stack([ii, ii, hh, hh], axis=1).reshape(-1)
    ccol = np.stack([ii, hh, hh, ii], axis=1).reshape(-1)
    ctab = _pack_pairs(cr, ccol)
    return np.concatenate([main, ctab])


_IDX_PACKED = _build_idx_table()


def _sqrl_gather_body(x_hbm, idx_hbm, out_hbm, idx_v, plane_v, corn_v,
                      outb_v, insem, osem):
    wid = lax.axis_index("s") * 2 + lax.axis_index("c")
    pltpu.sync_copy(idx_hbm, idx_v)
    iota16 = lax.iota(jnp.int32, 16)
    iota17 = iota16 * 17

    def unpack4(vp):
        r0 = jnp.bitwise_and(vp, 0xFF)
        c0 = jnp.bitwise_and(lax.shift_right_logical(vp, 8), 0xFF)
        r1 = jnp.bitwise_and(lax.shift_right_logical(vp, 16), 0xFF)
        c1 = lax.shift_right_logical(vp, 24)
        return r0, c0, r1, c1

    def drain_chunk(buf):
        pltpu.make_async_copy(
            out_hbm.at[0, 0, pl.ds(0, CROWS), :], outb_v.at[buf], osem.at[buf]
        ).wait()

    def plane_body(pi, carry):
        p = wid * PER_WORKER + pi
        pb = lax.div(p, C)
        pc = lax.rem(p, C)
        pltpu.async_copy(x_hbm.at[pb, pc], plane_v, insem).wait()

        def corner_body(kk, carry2):
            vp = idx_v[pl.ds(IDXWORDS + kk * 16, 16)]
            r0, c0, r1, c1 = unpack4(vp)
            g0 = plsc.load_gather(plane_v, [r0, c0])
            g1 = plsc.load_gather(plane_v, [r1, c1])
            base = kk * (32 * 17)
            for o in range(16):
                plsc.store_scatter(corn_v, [iota17 + (base + o)], g0)
                plsc.store_scatter(corn_v, [iota17 + (base + 272 + o)], g1)
            return carry2

        lax.fori_loop(0, CTABWORDS // 16, corner_body, 0)

        def chunk_body(c, carry2):
            buf = lax.rem(c, 2)

            @pl.when(c >= 2)
            def _():
                drain_chunk(buf)

            @plsc.parallel_loop(0, CROWS, unroll=1)
            def vbody(row):
                base = (c * (CROWS * ROWVREG) + row * ROWVREG) * 16
                arow = 68 * (c * CROWS + row) - 17 * H

                def emit(rf, cf, dst_off, kk):
                    rep = rf >= H
                    rc = jnp.minimum(rf, H - 1)
                    a_raw = rf * 17 + (arow + cf)
                    a = jnp.where(rep, a_raw, iota16)
                    g1 = plsc.load_gather(plane_v, [rc, cf])
                    g2 = plsc.load_gather(corn_v, [a])
                    outb_v[buf, row, pl.ds(kk * 32 + dst_off, 16)] = (
                        jnp.where(rep, g2, g1))

                for kk in range(ROWVREG):
                    vp = idx_v[pl.ds(base + kk * 16, 16)]
                    r0, c0, r1, c1 = unpack4(vp)
                    emit(r0, c0, 0, kk)
                    emit(r1, c1, 16, kk)

            pltpu.async_copy(
                outb_v.at[buf],
                out_hbm.at[pb, pc, pl.ds(c * CROWS, CROWS), :],
                osem.at[buf])
            return carry2

        lax.fori_loop(0, NCHUNK, chunk_body, 0)
        drain_chunk(0)
        drain_chunk(1)
        return carry

    lax.fori_loop(0, PER_WORKER, plane_body, 0)


@functools.cache
def _sqrl_gather():
    mesh = plsc.VectorSubcoreMesh(core_axis_name="c", subcore_axis_name="s")
    return pl.kernel(
        _sqrl_gather_body,
        out_type=jax.ShapeDtypeStruct((B, C, HH, OW), jnp.float32),
        mesh=mesh,
        scratch_types=[
            pltpu.VMEM((IDXWORDS + CTABWORDS,), jnp.int32),
            pltpu.VMEM((H, H), jnp.float32),
            pltpu.VMEM((CORNWORDS,), jnp.float32),
            pltpu.VMEM((2, CROWS, OW), jnp.float32),
            pltpu.SemaphoreType.DMA,
            pltpu.SemaphoreType.DMA((2,)),
        ],
        compiler_params=pltpu.CompilerParams(needs_layout_passes=False),
    )


def kernel(x):
    return _sqrl_gather()(x, jnp.asarray(_IDX_PACKED))


# speedup vs baseline: 2.1692x; 2.1692x over previous
import functools

import numpy as np
import jax
import jax.numpy as jnp
from jax import lax
from jax.experimental import pallas as pl
from jax.experimental.pallas import tpu as pltpu
from jax.experimental.pallas import tpu_sc as plsc

H = 224
HH = H // 2
OW = 4 * H
B = 4
C = 192
NPLANES = B * C
OUT_PLANE = HH * OW
NWORKERS = 32
PER_WORKER = NPLANES // NWORKERS
CROWS = 8
NCHUNK = HH // CROWS
CHUNK = CROWS * OW
ROWVREG = OW // 32
IDXWORDS = OUT_PLANE // 2


def _build_src_map() -> np.ndarray:
    lmid = (H - 1) // 2
    r = np.arange(HH)[:, None]
    j = np.arange(OW)[None, :]
    i = lmid - r
    el = 2 * r + 1
    hi = i + el
    b1 = 3 * i + el
    b2 = 3 * i + 2 * el
    b3 = 7 * i + 3 * el
    b4 = 7 * i + 4 * el
    body = 4 * H - 4
    k = 5 * i + 2 * el + hi
    src_a = i * H + np.clip(j - body * (j >= b4), i, hi)
    src_b = hi * H + np.clip(k - j, i, hi)
    src_cr = (j - (2 * i + el)) * H + hi
    src_cl = (body - j) * H + i
    src = np.where(j < b1, src_a,
          np.where(j < b2, src_cr,
          np.where(j < b3, src_b,
          np.where(j < b4, src_cl, src_a))))
    return src.reshape(-1)


def _build_packed_idx() -> np.ndarray:
    flat = _build_src_map().astype(np.uint32).reshape(-1, 2, 16)
    r0, c0 = flat[:, 0, :] // H, flat[:, 0, :] % H
    r1, c1 = flat[:, 1, :] // H, flat[:, 1, :] % H
    packed = r0 | (c0 << 8) | (r1 << 16) | (c1 << 24)
    return packed.reshape(-1).view(np.int32)


_IDX_PACKED = _build_packed_idx()


def _sqrl_gather_body(x_hbm, idx_hbm, out_hbm, idx_v, plane_v, outb_v,
                      insem, osem):
    wid = lax.axis_index("s") * 2 + lax.axis_index("c")
    pltpu.sync_copy(idx_hbm, idx_v)

    def drain_chunk(buf):
        pltpu.make_async_copy(
            out_hbm.at[0, 0, pl.ds(0, CROWS), :], outb_v.at[buf], osem.at[buf]
        ).wait()

    def plane_body(pi, carry):
        p = wid * PER_WORKER + pi
        pb = lax.div(p, C)
        pc = lax.rem(p, C)
        pltpu.async_copy(x_hbm.at[pb, pc], plane_v, insem).wait()

        def chunk_body(c, carry2):
            buf = lax.rem(c, 2)

            @pl.when(c >= 2)
            def _():
                drain_chunk(buf)

            @plsc.parallel_loop(0, CROWS, unroll=1)
            def vbody(row):
                base = (c * (CROWS * ROWVREG) + row * ROWVREG) * 16
                for kk in range(ROWVREG):
                    vp = idx_v[pl.ds(base + kk * 16, 16)]
                    r0 = jnp.bitwise_and(vp, 0xFF)
                    c0 = jnp.bitwise_and(lax.shift_right_logical(vp, 8), 0xFF)
                    r1 = jnp.bitwise_and(lax.shift_right_logical(vp, 16), 0xFF)
                    c1 = lax.shift_right_logical(vp, 24)
                    outb_v[buf, row, pl.ds(kk * 32, 16)] = (
                        plsc.load_gather(plane_v, [r0, c0]))
                    outb_v[buf, row, pl.ds(kk * 32 + 16, 16)] = (
                        plsc.load_gather(plane_v, [r1, c1]))

            pltpu.async_copy(
                outb_v.at[buf],
                out_hbm.at[pb, pc, pl.ds(c * CROWS, CROWS), :],
                osem.at[buf])
            return carry2

        lax.fori_loop(0, NCHUNK, chunk_body, 0)
        drain_chunk(0)
        drain_chunk(1)
        return carry

    lax.fori_loop(0, PER_WORKER, plane_body, 0)


@functools.cache
def _sqrl_gather():
    mesh = plsc.VectorSubcoreMesh(core_axis_name="c", subcore_axis_name="s")
    return pl.kernel(
        _sqrl_gather_body,
        out_type=jax.ShapeDtypeStruct((B, C, HH, OW), jnp.float32),
        mesh=mesh,
        scratch_types=[
            pltpu.VMEM((IDXWORDS,), jnp.int32),
            pltpu.VMEM((H, H), jnp.float32),
            pltpu.VMEM((2, CROWS, OW), jnp.float32),
            pltpu.SemaphoreType.DMA,
            pltpu.SemaphoreType.DMA((2,)),
        ],
        compiler_params=pltpu.CompilerParams(
            needs_layout_passes=False, disable_bounds_checks=True),
    )


def kernel(x):
    return _sqrl_gather()(x, jnp.asarray(_IDX_PACKED))


# speedup vs baseline: 3.7815x; 1.7433x over previous
import functools

import numpy as np
import jax
import jax.numpy as jnp
from jax import lax
from jax.experimental import pallas as pl
from jax.experimental.pallas import tpu as pltpu
from jax.experimental.pallas import tpu_sc as plsc

H = 224
HH = H // 2
OW = 4 * H
B = 4
C = 192
NPLANES = B * C
OUT_PLANE = HH * OW
NWORKERS = 32
PER_WORKER = NPLANES // NWORKERS
CROWS = 8
NCHUNK = HH // CROWS
CHUNK = CROWS * OW
ROWVREG = OW // 32
IDXWORDS = OUT_PLANE // 2
IDXCHUNK = IDXWORDS // NCHUNK
LSTRIDE = H + 1
LINWORDS = H * LSTRIDE


def _build_src_map() -> np.ndarray:
    lmid = (H - 1) // 2
    r = np.arange(HH)[:, None]
    j = np.arange(OW)[None, :]
    i = lmid - r
    el = 2 * r + 1
    hi = i + el
    b1 = 3 * i + el
    b2 = 3 * i + 2 * el
    b3 = 7 * i + 3 * el
    b4 = 7 * i + 4 * el
    body = 4 * H - 4
    k = 5 * i + 2 * el + hi
    src_a = i * H + np.clip(j - body * (j >= b4), i, hi)
    src_b = hi * H + np.clip(k - j, i, hi)
    src_cr = (j - (2 * i + el)) * H + hi
    src_cl = (body - j) * H + i
    src = np.where(j < b1, src_a,
          np.where(j < b2, src_cr,
          np.where(j < b3, src_b,
          np.where(j < b4, src_cl, src_a))))
    return src.reshape(-1)


def _build_packed_idx() -> np.ndarray:
    src = _build_src_map()
    phys = ((src // H) * LSTRIDE + src % H).astype(np.uint32).reshape(-1, 2, 16)
    packed = phys[:, 0, :] | (phys[:, 1, :] << 16)
    return packed.reshape(-1).view(np.int32)


_IDX_PACKED = _build_packed_idx()


def _sqrl_gather_body(x_hbm, idx_hbm, out_hbm, plane2_v, plane1_v, idxb_v,
                      outb_v, insem, isem, osem):
    wid = lax.axis_index("s") * 2 + lax.axis_index("c")

    def plane_dma(p, sync=False):
        pb = lax.div(p, C)
        pc = lax.rem(p, C)
        return pltpu.async_copy(x_hbm.at[pb, pc], plane2_v, insem)

    def idx_prefetch(c):
        pltpu.async_copy(idx_hbm.at[pl.ds(c * IDXCHUNK, IDXCHUNK)],
                         idxb_v.at[lax.rem(c, 2)], isem.at[lax.rem(c, 2)])

    def wait_input():
        pltpu.make_async_copy(x_hbm.at[0, 0], plane2_v, insem).wait()

    def wait_idx(buf):
        pltpu.make_async_copy(idx_hbm.at[pl.ds(0, IDXCHUNK)],
                              idxb_v.at[buf], isem.at[buf]).wait()

    def drain_out(buf):
        pltpu.make_async_copy(out_hbm.at[0, 0, pl.ds(0, CROWS), :],
                              outb_v.at[buf], osem.at[buf]).wait()

    plane_dma(wid * PER_WORKER)
    idx_prefetch(0)

    def plane_body(pi, carry):
        p = wid * PER_WORKER + pi
        pb = lax.div(p, C)
        pc = lax.rem(p, C)
        wait_input()

        @plsc.parallel_loop(0, H, unroll=1)
        def copy_row(row):
            for k in range(H // 16):
                plane1_v[pl.ds(row * LSTRIDE + k * 16, 16)] = (
                    plane2_v[row, pl.ds(k * 16, 16)])

        plane_dma(jnp.minimum(p + 1, NPLANES - 1))

        def chunk_body(c, carry2):
            buf = lax.rem(c, 2)
            wait_idx(buf)
            idx_prefetch(lax.rem(c + 1, NCHUNK))

            @pl.when(c >= 2)
            def _():
                drain_out(buf)

            @plsc.parallel_loop(0, CROWS, unroll=1)
            def vbody(row):
                base = row * (ROWVREG * 16)
                for kk in range(ROWVREG):
                    vp = idxb_v[buf, pl.ds(base + kk * 16, 16)]
                    lo = jnp.bitwise_and(vp, 0xFFFF)
                    hi = lax.shift_right_logical(vp, 16)
                    outb_v[buf, row, pl.ds(kk * 32, 16)] = (
                        plsc.load_gather(plane1_v, [lo]))
                    outb_v[buf, row, pl.ds(kk * 32 + 16, 16)] = (
                        plsc.load_gather(plane1_v, [hi]))

            pltpu.async_copy(
                outb_v.at[buf],
                out_hbm.at[pb, pc, pl.ds(c * CROWS, CROWS), :],
                osem.at[buf])
            return carry2

        lax.fori_loop(0, NCHUNK, chunk_body, 0)
        drain_out(0)
        drain_out(1)
        return carry

    lax.fori_loop(0, PER_WORKER, plane_body, 0)
    wait_input()
    wait_idx(0)


@functools.cache
def _sqrl_gather():
    mesh = plsc.VectorSubcoreMesh(core_axis_name="c", subcore_axis_name="s")
    return pl.kernel(
        _sqrl_gather_body,
        out_type=jax.ShapeDtypeStruct((B, C, HH, OW), jnp.float32),
        mesh=mesh,
        scratch_types=[
            pltpu.VMEM((H, H), jnp.float32),
            pltpu.VMEM((LINWORDS,), jnp.float32),
            pltpu.VMEM((2, IDXCHUNK), jnp.int32),
            pltpu.VMEM((2, CROWS, OW), jnp.float32),
            pltpu.SemaphoreType.DMA,
            pltpu.SemaphoreType.DMA((2,)),
            pltpu.SemaphoreType.DMA((2,)),
        ],
        compiler_params=pltpu.CompilerParams(
            needs_layout_passes=False, disable_bounds_checks=True),
    )


def kernel(x):
    return _sqrl_gather()(x, jnp.asarray(_IDX_PACKED))


# speedup vs baseline: 3.8311x; 1.0131x over previous
import functools

import numpy as np
import jax
import jax.numpy as jnp
from jax import lax
from jax.experimental import pallas as pl
from jax.experimental.pallas import tpu as pltpu
from jax.experimental.pallas import tpu_sc as plsc

H = 224
HH = H // 2
OW = 4 * H
B = 4
C = 192
NPLANES = B * C
OUT_PLANE = HH * OW
NWORKERS = 32
PER_WORKER = NPLANES // NWORKERS
CROWS = 8
NCHUNK = HH // CROWS
CHUNK = CROWS * OW
ROWVREG = OW // 32
IDXWORDS = OUT_PLANE // 2
IDXCHUNK = IDXWORDS // NCHUNK
LSTRIDE = H + 1
LINWORDS = H * LSTRIDE


def _build_src_map() -> np.ndarray:
    lmid = (H - 1) // 2
    r = np.arange(HH)[:, None]
    j = np.arange(OW)[None, :]
    i = lmid - r
    el = 2 * r + 1
    hi = i + el
    b1 = 3 * i + el
    b2 = 3 * i + 2 * el
    b3 = 7 * i + 3 * el
    b4 = 7 * i + 4 * el
    body = 4 * H - 4
    k = 5 * i + 2 * el + hi
    src_a = i * H + np.clip(j - body * (j >= b4), i, hi)
    src_b = hi * H + np.clip(k - j, i, hi)
    src_cr = (j - (2 * i + el)) * H + hi
    src_cl = (body - j) * H + i
    src = np.where(j < b1, src_a,
          np.where(j < b2, src_cr,
          np.where(j < b3, src_b,
          np.where(j < b4, src_cl, src_a))))
    return src.reshape(-1)


def _build_packed_idx() -> np.ndarray:
    src = _build_src_map()
    phys = ((src // H) * LSTRIDE + src % H).astype(np.uint32).reshape(-1, 2, 16)
    packed = phys[:, 0, :] | (phys[:, 1, :] << 16)
    return packed.reshape(-1).view(np.int32)


_IDX_PACKED = _build_packed_idx()


def _sqrl_gather_body(x_hbm, idx_hbm, out_hbm, plane2_v, plane1_v, idxb_v,
                      outb_v, insem, isem, osem):
    wid = lax.axis_index("s") * 2 + lax.axis_index("c")

    def plane_dma(p, sync=False):
        pb = lax.div(p, C)
        pc = lax.rem(p, C)
        return pltpu.async_copy(x_hbm.at[pb, pc], plane2_v, insem)

    def idx_prefetch(c):
        pltpu.async_copy(idx_hbm.at[pl.ds(c * IDXCHUNK, IDXCHUNK)],
                         idxb_v.at[lax.rem(c, 2)], isem.at[lax.rem(c, 2)])

    def wait_input():
        pltpu.make_async_copy(x_hbm.at[0, 0], plane2_v, insem).wait()

    def wait_idx(buf):
        pltpu.make_async_copy(idx_hbm.at[pl.ds(0, IDXCHUNK)],
                              idxb_v.at[buf], isem.at[buf]).wait()

    def drain_out(buf):
        pltpu.make_async_copy(out_hbm.at[0, 0, pl.ds(0, CROWS), :],
                              outb_v.at[buf], osem.at[buf]).wait()

    plane_dma(wid * PER_WORKER)
    idx_prefetch(0)

    def plane_body(pi, carry):
        p = wid * PER_WORKER + pi
        pb = lax.div(p, C)
        pc = lax.rem(p, C)
        wait_input()

        @plsc.parallel_loop(0, H, unroll=2)
        def copy_row(row):
            for k in range(H // 16):
                plane1_v[pl.ds(row * LSTRIDE + k * 16, 16)] = (
                    plane2_v[row, pl.ds(k * 16, 16)])

        plane_dma(jnp.minimum(p + 1, NPLANES - 1))

        def chunk_body(c, carry2):
            buf = lax.rem(c, 2)
            wait_idx(buf)
            idx_prefetch(lax.rem(c + 1, NCHUNK))

            @pl.when(c >= 2)
            def _():
                drain_out(buf)

            @plsc.parallel_loop(0, CROWS, unroll=2)
            def vbody(row):
                base = row * (ROWVREG * 16)
                for kk in range(ROWVREG):
                    vp = idxb_v[buf, pl.ds(base + kk * 16, 16)]
                    lo = jnp.bitwise_and(vp, 0xFFFF)
                    hi = lax.shift_right_logical(vp, 16)
                    outb_v[buf, row, pl.ds(kk * 32, 16)] = (
                        plsc.load_gather(plane1_v, [lo]))
                    outb_v[buf, row, pl.ds(kk * 32 + 16, 16)] = (
                        plsc.load_gather(plane1_v, [hi]))

            pltpu.async_copy(
                outb_v.at[buf],
                out_hbm.at[pb, pc, pl.ds(c * CROWS, CROWS), :],
                osem.at[buf])
            return carry2

        lax.fori_loop(0, NCHUNK, chunk_body, 0)
        drain_out(0)
        drain_out(1)
        return carry

    lax.fori_loop(0, PER_WORKER, plane_body, 0)
    wait_input()
    wait_idx(0)


@functools.cache
def _sqrl_gather():
    mesh = plsc.VectorSubcoreMesh(core_axis_name="c", subcore_axis_name="s")
    return pl.kernel(
        _sqrl_gather_body,
        out_type=jax.ShapeDtypeStruct((B, C, HH, OW), jnp.float32),
        mesh=mesh,
        scratch_types=[
            pltpu.VMEM((H, H), jnp.float32),
            pltpu.VMEM((LINWORDS,), jnp.float32),
            pltpu.VMEM((2, IDXCHUNK), jnp.int32),
            pltpu.VMEM((2, CROWS, OW), jnp.float32),
            pltpu.SemaphoreType.DMA,
            pltpu.SemaphoreType.DMA((2,)),
            pltpu.SemaphoreType.DMA((2,)),
        ],
        compiler_params=pltpu.CompilerParams(
            needs_layout_passes=False, disable_bounds_checks=True),
    )


def kernel(x):
    return _sqrl_gather()(x, jnp.asarray(_IDX_PACKED))
